# Initial kernel scaffold; baseline (speedup 1.0000x reference)
import jax
import jax.numpy as jnp
from jax.experimental import pallas as pl


def kernel(x, edge_index, weight):
    raise NotImplementedError("write your pallas kernel here")


# speedup vs baseline: 1.0766x; 1.0766x over previous
import jax
import jax.numpy as jnp
from jax import lax
from jax.experimental import pallas as pl
from jax.experimental.pallas import tpu as pltpu

N = 100000
D = 128
K = 50000
B = 8192
G = 13
RB = B // 128
R = 1024
C = 128
LOGP = 17
OUTR = 392
IMIN = -2147483648


def _roll(a, shift, axis):
    size = R if axis == 0 else C
    return pltpu.roll(a, shift % size, axis)


def _body(x_ref, w_ref, norm_ref, idx_out, val_out, sc_ref):
    s = pl.program_id(0)
    xb = x_ref[...].astype(jnp.bfloat16)
    raw = lax.dot_general(xb, w_ref[...], (((1,), (0,)), ((), ())),
                          preferred_element_type=jnp.float32)
    sc_ref[pl.ds(s * RB, RB), :] = raw[:, 0:1].reshape(RB, C)

    @pl.when(s == G - 1)
    def _sort():
        norm = norm_ref[0, 0]
        val = jnp.tanh(sc_ref[...] / (norm + 1e-16))
        b = lax.bitcast_convert_type(val, jnp.int32)
        key = jnp.where(b < 0, b ^ jnp.int32(0x7FFFFFFF), b)
        row = lax.broadcasted_iota(jnp.int32, (R, C), 0)
        col = lax.broadcasted_iota(jnp.int32, (R, C), 1)
        idx = row * C + col
        key = jnp.where(idx < N, key, IMIN)

        def stage(key, idx, k, j):
            d = 1 << j
            if j < 7:
                axis, dd = 1, d
                hi = (col & d) != 0
            else:
                axis, dd = 0, d >> 7
                hi = (row & dd) != 0
            pk = jnp.where(hi, _roll(key, dd, axis), _roll(key, -dd, axis))
            pi = jnp.where(hi, _roll(idx, dd, axis), _roll(idx, -dd, axis))
            if k < 7:
                dirbit = (col >> k) & 1
            else:
                dirbit = (row >> (k - 7)) & 1
            dir_ = dirbit == 0
            a_before = (key > pk) | ((key == pk) & (idx < pi))
            want_before = hi ^ dir_
            keep = a_before == want_before
            return jnp.where(keep, key, pk), jnp.where(keep, idx, pi)

        for k in range(1, LOGP + 1):
            for j in range(k - 1, -1, -1):
                key, idx = stage(key, idx, k, j)

        idx_out[...] = idx[:OUTR, :]
        kb = key[:OUTR, :]
        kb = jnp.where(kb < 0, kb ^ jnp.int32(0x7FFFFFFF), kb)
        val_out[...] = lax.bitcast_convert_type(kb, jnp.float32)


def _run(x, wp, norm):
    return pl.pallas_call(
        _body,
        grid=(G,),
        in_specs=[
            pl.BlockSpec((B, D), lambda s: (s, 0)),
            pl.BlockSpec((D, C), lambda s: (0, 0)),
            pl.BlockSpec((1, 1), lambda s: (0, 0)),
        ],
        out_specs=[
            pl.BlockSpec((OUTR, C), lambda s: (0, 0)),
            pl.BlockSpec((OUTR, C), lambda s: (0, 0)),
        ],
        out_shape=[
            jax.ShapeDtypeStruct((OUTR, C), jnp.int32),
            jax.ShapeDtypeStruct((OUTR, C), jnp.float32),
        ],
        scratch_shapes=[pltpu.VMEM((R, C), jnp.float32)],
    )(x, wp, norm)


def kernel(x, edge_index, weight):
    norm = jnp.linalg.norm(weight, ord=2).reshape(1, 1)
    wcol = weight.reshape(D, 1).astype(jnp.bfloat16)
    wp = jnp.pad(wcol, ((0, 0), (0, C - 1)))
    idx2d, val2d = _run(x, wp, norm)
    return (idx2d.reshape(-1)[:K], val2d.reshape(-1)[:K])


# speedup vs baseline: 1.1885x; 1.1040x over previous
import jax
import jax.numpy as jnp
from jax import lax
from jax.experimental import pallas as pl
from jax.experimental.pallas import tpu as pltpu

N = 100000
D = 128
K = 50000
B = 32768
G = 4
RB = B // 128
R = 1024
C = 128
LOGP = 17
OUTR = 392
IMIN = -2147483648


def _roll(a, shift, axis):
    size = R if axis == 0 else C
    return pltpu.roll(a, shift % size, axis)


def _body(x_ref, w_ref, norm_ref, idx_out, val_out, sc_ref):
    s = pl.program_id(0)
    xb = x_ref[...].astype(jnp.bfloat16)
    raw = lax.dot_general(xb, w_ref[...], (((1,), (0,)), ((), ())),
                          preferred_element_type=jnp.float32)
    sc_ref[pl.ds(s * RB, RB), :] = raw[:, 0:1].reshape(RB, C)

    @pl.when(s == G - 1)
    def _sort():
        norm = norm_ref[0, 0]
        val = jnp.tanh(sc_ref[...] / (norm + 1e-16))
        b = lax.bitcast_convert_type(val, jnp.int32)
        key = jnp.where(b < 0, b ^ jnp.int32(0x7FFFFFFF), b)
        row = lax.broadcasted_iota(jnp.int32, (R, C), 0)
        col = lax.broadcasted_iota(jnp.int32, (R, C), 1)
        idx = row * C + col
        key = jnp.where(idx < N, key, IMIN)

        def stage(key, idx, k, j):
            d = 1 << j
            if j < 7:
                axis, dd = 1, d
                hi = (col & d) != 0
            else:
                axis, dd = 0, d >> 7
                hi = (row & dd) != 0
            pk = jnp.where(hi, _roll(key, dd, axis), _roll(key, -dd, axis))
            pi = jnp.where(hi, _roll(idx, dd, axis), _roll(idx, -dd, axis))
            if k < 7:
                dirbit = (col >> k) & 1
            else:
                dirbit = (row >> (k - 7)) & 1
            dir_ = dirbit == 0
            a_before = (key > pk) | ((key == pk) & (idx < pi))
            want_before = hi ^ dir_
            keep = a_before == want_before
            return jnp.where(keep, key, pk), jnp.where(keep, idx, pi)

        for k in range(1, LOGP + 1):
            for j in range(k - 1, -1, -1):
                key, idx = stage(key, idx, k, j)

        idx_out[...] = idx[:OUTR, :]
        kb = key[:OUTR, :]
        kb = jnp.where(kb < 0, kb ^ jnp.int32(0x7FFFFFFF), kb)
        val_out[...] = lax.bitcast_convert_type(kb, jnp.float32)


def _run(x, wp, norm):
    return pl.pallas_call(
        _body,
        grid=(G,),
        in_specs=[
            pl.BlockSpec((B, D), lambda s: (s, 0)),
            pl.BlockSpec((D, C), lambda s: (0, 0)),
            pl.BlockSpec((1, 1), lambda s: (0, 0)),
        ],
        out_specs=[
            pl.BlockSpec((OUTR, C), lambda s: (0, 0)),
            pl.BlockSpec((OUTR, C), lambda s: (0, 0)),
        ],
        out_shape=[
            jax.ShapeDtypeStruct((OUTR, C), jnp.int32),
            jax.ShapeDtypeStruct((OUTR, C), jnp.float32),
        ],
        scratch_shapes=[pltpu.VMEM((R, C), jnp.float32)],
    )(x, wp, norm)


def kernel(x, edge_index, weight):
    norm = jnp.linalg.norm(weight, ord=2).reshape(1, 1)
    wcol = weight.reshape(D, 1).astype(jnp.bfloat16)
    wp = jnp.pad(wcol, ((0, 0), (0, C - 1)))
    idx2d, val2d = _run(x, wp, norm)
    return (idx2d.reshape(-1)[:K], val2d.reshape(-1)[:K])


# speedup vs baseline: 1.2608x; 1.0608x over previous
import jax
import jax.numpy as jnp
from jax import lax
from jax.experimental import pallas as pl
from jax.experimental.pallas import tpu as pltpu

N = 100000
D = 128
K = 50000
B = 32768
G = 4
RB = B // 128
R = 1024
C = 128
LOGP = 17
OUTC = 64
IMIN = -2147483648


def _roll(a, shift, axis):
    size = R if axis == 0 else C
    return pltpu.roll(a, shift % size, axis)


def _body(x_ref, w_ref, norm_ref, idx_out, val_out, sc_ref):
    s = pl.program_id(0)
    xb = x_ref[...].astype(jnp.bfloat16)
    raw = lax.dot_general(xb, w_ref[...], (((1,), (0,)), ((), ())),
                          preferred_element_type=jnp.float32)
    sc_ref[pl.ds(s * RB, RB), :] = raw[:, 0:1].reshape(RB, C)

    @pl.when(s == G - 1)
    def _sort():
        norm = norm_ref[0, 0]
        val = jnp.tanh(sc_ref[...] / (norm + 1e-16))
        b = lax.bitcast_convert_type(val, jnp.int32)
        key = jnp.where(b < 0, b ^ jnp.int32(0x7FFFFFFF), b)
        row = lax.broadcasted_iota(jnp.int32, (R, C), 0)
        col = lax.broadcasted_iota(jnp.int32, (R, C), 1)
        idx = row * C + col
        key = jnp.where(idx < N, key, IMIN)

        def stage(key, idx, k, j):
            if j < 10:
                axis, dd = 0, 1 << j
                hi = (row & dd) != 0
            else:
                axis, dd = 1, 1 << (j - 10)
                hi = (col & dd) != 0
            pk = jnp.where(hi, _roll(key, dd, axis), _roll(key, -dd, axis))
            pi = jnp.where(hi, _roll(idx, dd, axis), _roll(idx, -dd, axis))
            if k < 10:
                dirbit = (row >> k) & 1
            else:
                dirbit = (col >> (k - 10)) & 1
            dir_ = dirbit == 0
            a_before = (key > pk) | ((key == pk) & (idx < pi))
            want_before = hi ^ dir_
            keep = a_before == want_before
            return jnp.where(keep, key, pk), jnp.where(keep, idx, pi)

        for k in range(1, LOGP + 1):
            for j in range(k - 1, -1, -1):
                key, idx = stage(key, idx, k, j)

        idx_out[...] = idx[:, :OUTC]
        kb = key[:, :OUTC]
        kb = jnp.where(kb < 0, kb ^ jnp.int32(0x7FFFFFFF), kb)
        val_out[...] = lax.bitcast_convert_type(kb, jnp.float32)


def _run(x, wp, norm):
    return pl.pallas_call(
        _body,
        grid=(G,),
        in_specs=[
            pl.BlockSpec((B, D), lambda s: (s, 0)),
            pl.BlockSpec((D, C), lambda s: (0, 0)),
            pl.BlockSpec((1, 1), lambda s: (0, 0)),
        ],
        out_specs=[
            pl.BlockSpec((R, OUTC), lambda s: (0, 0)),
            pl.BlockSpec((R, OUTC), lambda s: (0, 0)),
        ],
        out_shape=[
            jax.ShapeDtypeStruct((R, OUTC), jnp.int32),
            jax.ShapeDtypeStruct((R, OUTC), jnp.float32),
        ],
        scratch_shapes=[pltpu.VMEM((R, C), jnp.float32)],
    )(x, wp, norm)


def kernel(x, edge_index, weight):
    norm = jnp.linalg.norm(weight, ord=2).reshape(1, 1)
    wcol = weight.reshape(D, 1).astype(jnp.bfloat16)
    wp = jnp.pad(wcol, ((0, 0), (0, C - 1)))
    idx2d, val2d = _run(x, wp, norm)
    return (idx2d.T.reshape(-1)[:K], val2d.T.reshape(-1)[:K])


# speedup vs baseline: 1.2807x; 1.0158x over previous
import jax
import jax.numpy as jnp
from jax import lax
from jax.experimental import pallas as pl
from jax.experimental.pallas import tpu as pltpu

N = 100000
D = 128
K = 50000
B = 32768
G = 4
RB = B // 128
R = 1024
C = 128
LOGP = 17
OUTC = 64
IMIN = -2147483648


def _roll(a, shift, axis):
    size = R if axis == 0 else C
    return pltpu.roll(a, shift % size, axis)


def _body(x_ref, w_ref, norm_ref, idx_out, val_out, sc_ref):
    s = pl.program_id(0)
    xb = x_ref[...].astype(jnp.bfloat16)
    raw = lax.dot_general(xb, w_ref[...], (((1,), (0,)), ((), ())),
                          preferred_element_type=jnp.float32)
    sc_ref[pl.ds(s * RB, RB), :] = raw[:, 0:1].reshape(RB, C)

    @pl.when(s == G - 1)
    def _sort():
        norm = norm_ref[0, 0]
        val = jnp.tanh(sc_ref[...] / (norm + 1e-16))
        b = lax.bitcast_convert_type(val, jnp.int32)
        key = jnp.where(b < 0, b ^ jnp.int32(0x7FFFFFFF), b)
        row = lax.broadcasted_iota(jnp.int32, (R, C), 0)
        col = lax.broadcasted_iota(jnp.int32, (R, C), 1)
        idx = row * C + col
        key = jnp.where(idx < N, key, IMIN)

        bits = [(row & (1 << b)) != 0 for b in range(10)] + \
               [(col & (1 << b)) != 0 for b in range(7)]

        def stage(key, idx, k, j):
            if j < 10:
                axis, dd = 0, 1 << j
            else:
                axis, dd = 1, 1 << (j - 10)
            hi = bits[j]
            pk = jnp.where(hi, _roll(key, dd, axis), _roll(key, -dd, axis))
            pi = jnp.where(hi, _roll(idx, dd, axis), _roll(idx, -dd, axis))
            a_before = (key > pk) | ((key == pk) & (idx < pi))
            want_before = (hi == bits[k]) if k < LOGP else ~hi
            keep = a_before == want_before
            return jnp.where(keep, key, pk), jnp.where(keep, idx, pi)

        for k in range(1, LOGP + 1):
            for j in range(k - 1, -1, -1):
                key, idx = stage(key, idx, k, j)

        idx_out[...] = idx[:, :OUTC]
        kb = key[:, :OUTC]
        kb = jnp.where(kb < 0, kb ^ jnp.int32(0x7FFFFFFF), kb)
        val_out[...] = lax.bitcast_convert_type(kb, jnp.float32)


def _run(x, wp, norm):
    return pl.pallas_call(
        _body,
        grid=(G,),
        in_specs=[
            pl.BlockSpec((B, D), lambda s: (s, 0)),
            pl.BlockSpec((D, C), lambda s: (0, 0)),
            pl.BlockSpec((1, 1), lambda s: (0, 0)),
        ],
        out_specs=[
            pl.BlockSpec((R, OUTC), lambda s: (0, 0)),
            pl.BlockSpec((R, OUTC), lambda s: (0, 0)),
        ],
        out_shape=[
            jax.ShapeDtypeStruct((R, OUTC), jnp.int32),
            jax.ShapeDtypeStruct((R, OUTC), jnp.float32),
        ],
        scratch_shapes=[pltpu.VMEM((R, C), jnp.float32)],
    )(x, wp, norm)


def kernel(x, edge_index, weight):
    norm = jnp.linalg.norm(weight, ord=2).reshape(1, 1)
    wcol = weight.reshape(D, 1).astype(jnp.bfloat16)
    wp = jnp.pad(wcol, ((0, 0), (0, C - 1)))
    idx2d, val2d = _run(x, wp, norm)
    return (idx2d.T.reshape(-1)[:K], val2d.T.reshape(-1)[:K])


# speedup vs baseline: 1.4632x; 1.1425x over previous
import jax
import jax.numpy as jnp
from jax import lax
from jax.experimental import pallas as pl
from jax.experimental.pallas import tpu as pltpu

N = 100000
D = 128
K = 50000
B = 32768
G = 4
BR = 256
R = 1024
C = 128
LOGP = 17
IMIN = -2147483648


def _roll(a, shift, axis):
    return pltpu.roll(a, shift % a.shape[axis], axis)


def _axis_dd(j):
    if j < 8:
        return 0, 1 << j
    if j < 15:
        return 1, 1 << (j - 8)
    return 0, 1 << (j - 7)


def _mkbits(rows):
    row = lax.broadcasted_iota(jnp.int32, (rows, C), 0)
    col = lax.broadcasted_iota(jnp.int32, (rows, C), 1)
    bits = []
    for b in range(LOGP):
        if b < 8:
            bits.append((row & (1 << b)) != 0)
        elif b < 15:
            bits.append((col & (1 << (b - 8))) != 0)
        else:
            bits.append((row & (1 << (b - 7))) != 0)
    return row, col, bits


def _stage(key, idx, j, hi, want):
    axis, dd = _axis_dd(j)
    pk = jnp.where(hi, _roll(key, dd, axis), _roll(key, -dd, axis))
    pi = jnp.where(hi, _roll(idx, dd, axis), _roll(idx, -dd, axis))
    a_before = (key > pk) | ((key == pk) & (idx < pi))
    keep = a_before == want
    return jnp.where(keep, key, pk), jnp.where(keep, idx, pi)


def _body(x_ref, w_ref, norm_ref, idx_out, val_out, key_ref, id_ref):
    s = pl.program_id(0)
    norm = norm_ref[0, 0]
    xb = x_ref[...].astype(jnp.bfloat16)
    raw = lax.dot_general(xb, w_ref[...], (((1,), (0,)), ((), ())),
                          preferred_element_type=jnp.float32)
    t = jnp.swapaxes(raw[:, 0:1].reshape(C, BR), 0, 1)
    val = jnp.tanh(t / (norm + 1e-16))
    bb = lax.bitcast_convert_type(val, jnp.int32)
    key = jnp.where(bb < 0, bb ^ jnp.int32(0x7FFFFFFF), bb)

    row, col, bits = _mkbits(BR)
    nid = s * B + col * BR + row
    key = jnp.where(nid < N, key, IMIN)
    idx = nid

    dir15 = (s & 1) == 0
    for k in range(1, 16):
        want_k = bits[k] if k < 15 else None
        for j in range(k - 1, -1, -1):
            hi = bits[j]
            if want_k is None:
                want = jnp.logical_xor(hi, dir15)
            else:
                want = hi == want_k
            key, idx = _stage(key, idx, j, hi, want)

    key_ref[pl.ds(s * BR, BR), :] = key
    id_ref[pl.ds(s * BR, BR), :] = idx

    @pl.when(s == G - 1)
    def _merge():
        kf = key_ref[...]
        if_ = id_ref[...]
        _, _, fb = _mkbits(R)
        for j in range(15, -1, -1):
            hi = fb[j]
            kf, if_ = _stage(kf, if_, j, hi, hi == fb[16])
        hi = fb[16]
        kf, if_ = _stage(kf, if_, 16, hi, jnp.logical_not(hi))
        kh = kf[:R // 2, :]
        ih = if_[:R // 2, :]
        _, _, hbits = _mkbits(R // 2)
        for j in range(15, -1, -1):
            hi = hbits[j]
            kh, ih = _stage(kh, ih, j, hi, jnp.logical_not(hi))
        idx_out[...] = ih
        kb = jnp.where(kh < 0, kh ^ jnp.int32(0x7FFFFFFF), kh)
        val_out[...] = lax.bitcast_convert_type(kb, jnp.float32)


def _run(x, wp, norm):
    return pl.pallas_call(
        _body,
        grid=(G,),
        in_specs=[
            pl.BlockSpec((B, D), lambda s: (s, 0)),
            pl.BlockSpec((D, C), lambda s: (0, 0)),
            pl.BlockSpec((1, 1), lambda s: (0, 0)),
        ],
        out_specs=[
            pl.BlockSpec((R // 2, C), lambda s: (0, 0)),
            pl.BlockSpec((R // 2, C), lambda s: (0, 0)),
        ],
        out_shape=[
            jax.ShapeDtypeStruct((R // 2, C), jnp.int32),
            jax.ShapeDtypeStruct((R // 2, C), jnp.float32),
        ],
        scratch_shapes=[pltpu.VMEM((R, C), jnp.int32),
                        pltpu.VMEM((R, C), jnp.int32)],
    )(x, wp, norm)


def kernel(x, edge_index, weight):
    norm = jnp.linalg.norm(weight, ord=2).reshape(1, 1)
    wcol = weight.reshape(D, 1).astype(jnp.bfloat16)
    wp = jnp.pad(wcol, ((0, 0), (0, C - 1)))
    idx2d, val2d = _run(x, wp, norm)
    ni = idx2d.reshape(2, BR, C).swapaxes(1, 2).reshape(-1)[:K]
    tv = val2d.reshape(2, BR, C).swapaxes(1, 2).reshape(-1)[:K]
    return (ni, tv)
